# baseline (speedup 1.0000x reference)
import dataclasses
import functools

import jax
import jax.numpy as jnp
import numpy as np
from jax import lax
from jax.experimental import pallas as pl
from jax.experimental.pallas import tpu as pltpu
from jax.experimental.pallas import tpu_sc as plsc

BATCH = 1024
NUM_GROUP = 2
TAU = 0.2
VD = 16 * 768
AD = 128
TD = 768

NC = 2
NS = 16
L = 16
NW = NC * NS
BPW = BATCH // NW
VCHUNK = 4
TCHUNK = 16


def _argmax_body(g_ref, group_ref, ta_ref, tb_ref, out_ref):
    grp = group_ref[...]
    c = jnp.sum(grp)
    a = ta_ref[c]
    b = tb_ref[c]
    logit = jnp.where(grp == 1, b, a)
    v = g_ref[...] + logit
    m = jnp.max(v, axis=1, keepdims=True)
    cols = lax.broadcasted_iota(jnp.int32, v.shape, 1)
    masked = jnp.where(v == m, cols, BATCH)
    out_ref[...] = jnp.min(masked, axis=1, keepdims=True)


def _tc_argmax(gumbel_field, group_row, table_a, table_b):
    return pl.pallas_call(
        _argmax_body,
        in_specs=[
            pl.BlockSpec(memory_space=pltpu.MemorySpace.VMEM),
            pl.BlockSpec(memory_space=pltpu.MemorySpace.VMEM),
            pl.BlockSpec(memory_space=pltpu.MemorySpace.SMEM),
            pl.BlockSpec(memory_space=pltpu.MemorySpace.SMEM),
        ],
        out_shape=jax.ShapeDtypeStruct((BATCH, 1), jnp.int32),
    )(gumbel_field, group_row, table_a, table_b)


GROUP_ROWS = 16


def _tc_video_body(fused_ref, video_ref, out_ref, vbuf0, vbuf1,
                   isem0, isem1, osem0, osem1):
    ngroups = BATCH // GROUP_ROWS
    bufs = (vbuf0, vbuf1)
    isems = (isem0, isem1)
    osems = (osem0, osem1)

    def start_in(g):
        b = g % 2
        handles = []
        for r in range(GROUP_ROWS):
            f = fused_ref[g * GROUP_ROWS + r]
            handles.append(pltpu.make_async_copy(
                video_ref.at[f], bufs[b].at[r], isems[b]))
        for h in handles:
            h.start()
        return handles

    def start_out(g):
        b = g % 2
        h = pltpu.make_async_copy(
            bufs[b], out_ref.at[pl.ds(g * GROUP_ROWS, GROUP_ROWS)], osems[b])
        h.start()
        return h

    in_h = start_in(0)
    out_h = [None] * ngroups
    for g in range(ngroups):
        for h in in_h:
            h.wait()
        out_h[g] = start_out(g)
        if g + 1 < ngroups:
            if g >= 1:
                out_h[g - 1].wait()
            in_h = start_in(g + 1)
    out_h[ngroups - 2].wait()
    out_h[ngroups - 1].wait()


def _tc_video_gather(fused, video):
    return pl.pallas_call(
        _tc_video_body,
        grid_spec=pltpu.PrefetchScalarGridSpec(
            num_scalar_prefetch=1,
            grid=(1,),
            in_specs=[pl.BlockSpec(memory_space=pl.ANY)],
            out_specs=pl.BlockSpec(memory_space=pl.ANY),
            scratch_shapes=[
                pltpu.VMEM((GROUP_ROWS, 16, 768), jnp.float32),
                pltpu.VMEM((GROUP_ROWS, 16, 768), jnp.float32),
                pltpu.SemaphoreType.DMA,
                pltpu.SemaphoreType.DMA,
                pltpu.SemaphoreType.DMA,
                pltpu.SemaphoreType.DMA,
            ],
        ),
        out_shape=jax.ShapeDtypeStruct((BATCH, 16, 768), jnp.float32),
    )(fused, video)


def _sc_gather_body(audio_hbm, text_hbm, tgt_hbm, grp_hbm, idx_hbm,
                    shuf_hbm, a_out, t_out, tg_out, gr_out,
                    idx_v, tgt_v, grp_v, shuf_v, fused_v, tgo_v, gro_v,
                    abuf, tbuf,
                    gsem0, gsem1, wsem0, wsem1):
    wid = lax.axis_index("s") * NC + lax.axis_index("c")
    base = wid * BPW

    h_idx = pltpu.async_copy(idx_hbm, idx_v, gsem0)
    h_shuf = pltpu.async_copy(shuf_hbm.at[pl.ds(base, BPW)], shuf_v, gsem1)
    h_tgt = pltpu.async_copy(tgt_hbm, tgt_v, wsem0)
    h_grp = pltpu.async_copy(grp_hbm, grp_v, wsem1)
    h_idx.wait()
    h_shuf.wait()
    h_tgt.wait()
    h_grp.wait()

    for k in range(0, BPW, L):
        sh = shuf_v[pl.ds(k, L)]
        f = plsc.load_gather(idx_v, [sh])
        fused_v[pl.ds(k, L)] = f
        tgo_v[pl.ds(k, L)] = plsc.load_gather(tgt_v, [f])
        gro_v[pl.ds(k, L)] = plsc.load_gather(grp_v, [f])

    pltpu.sync_copy(tgo_v, tg_out.at[pl.ds(base, BPW)])
    pltpu.sync_copy(gro_v, gr_out.at[pl.ds(base, BPW)])
    pltpu.sync_copy(audio_hbm.at[fused_v], abuf)
    pltpu.sync_copy(abuf, a_out.at[pl.ds(base, BPW)])
    for c in range(0, BPW, TCHUNK):
        pltpu.sync_copy(text_hbm.at[fused_v.at[pl.ds(c, TCHUNK)]], tbuf)
        pltpu.sync_copy(tbuf, t_out.at[pl.ds(base + c, TCHUNK)])


@functools.cache
def _sc_gather_kernel():
    mesh = plsc.VectorSubcoreMesh(core_axis_name="c", subcore_axis_name="s")
    cp = pltpu.CompilerParams()
    if "needs_layout_passes" in pltpu.CompilerParams.__dataclass_fields__:
        cp = dataclasses.replace(cp, needs_layout_passes=False)
    return pl.kernel(
        _sc_gather_body,
        compiler_params=cp,
        out_type=[
            jax.ShapeDtypeStruct((BATCH, AD), jnp.float32),
            jax.ShapeDtypeStruct((BATCH, TD), jnp.float32),
            jax.ShapeDtypeStruct((BATCH,), jnp.int32),
            jax.ShapeDtypeStruct((BATCH,), jnp.int32),
        ],
        mesh=mesh,
        scratch_types=[
            pltpu.VMEM((BATCH,), jnp.int32),
            pltpu.VMEM((BATCH,), jnp.int32),
            pltpu.VMEM((BATCH,), jnp.int32),
            pltpu.VMEM((BPW,), jnp.int32),
            pltpu.VMEM((BPW,), jnp.int32),
            pltpu.VMEM((BPW,), jnp.int32),
            pltpu.VMEM((BPW,), jnp.int32),
            pltpu.VMEM((BPW, AD), jnp.float32),
            pltpu.VMEM((TCHUNK, TD), jnp.float32),
            pltpu.SemaphoreType.DMA,
            pltpu.SemaphoreType.DMA,
            pltpu.SemaphoreType.DMA,
            pltpu.SemaphoreType.DMA,
        ],
    )


@functools.cache
def _fixed_draws():
    with jax.ensure_compile_time_eval():
        key = jax.random.key(42)
        k_mult, k_perm = jax.random.split(key)
        g = np.asarray(jax.random.gumbel(k_mult, (BATCH, BATCH), jnp.float32))
        shuf = np.asarray(jax.random.permutation(k_perm, BATCH), dtype=np.int32)
        c1 = jnp.arange(BATCH + 1, dtype=jnp.int32)
        wa0 = ((BATCH - c1).astype(jnp.float32) / BATCH) ** TAU
        wa1 = (c1.astype(jnp.float32) / BATCH) ** TAU
        s = wa0 + wa1
        table_a = np.asarray(jnp.log(wa0 / s))
        table_b = np.asarray(jnp.log(wa1 / s))
    invshuf = np.argsort(shuf).astype(np.int32)
    return g, shuf, invshuf, table_a, table_b


def kernel(batch_video, batch_audio, batch_text, batch_target, batch_group):
    g_np, shuf_np, _, ta_np, tb_np = _fixed_draws()
    gumbel_field = jnp.asarray(g_np)
    shuffle_idx = jnp.asarray(shuf_np)

    indices = _tc_argmax(gumbel_field, batch_group[None, :],
                         jnp.asarray(ta_np), jnp.asarray(tb_np)).reshape(BATCH)

    a, t, tg, gr = _sc_gather_kernel()(batch_audio, batch_text,
                                       batch_target, batch_group, indices,
                                       shuffle_idx)
    fused = indices[shuffle_idx]
    v = _tc_video_gather(fused, batch_video)
    return (v, a, t, tg, gr)


# speedup vs baseline: 1.9857x; 1.9857x over previous
import dataclasses
import functools

import jax
import jax.numpy as jnp
import numpy as np
from jax import lax
from jax.experimental import pallas as pl
from jax.experimental.pallas import tpu as pltpu
from jax.experimental.pallas import tpu_sc as plsc

BATCH = 1024
NUM_GROUP = 2
TAU = 0.2
VD = 16 * 768
AD = 128
TD = 768

NC = 2
NS = 16
L = 16
NW = NC * NS
BPW = BATCH // NW
VCHUNK = 4
TCHUNK = 16


def _argmax_body(g_ref, group_ref, ta_ref, tb_ref, out_ref):
    grp = group_ref[...]
    c = jnp.sum(grp)
    a = ta_ref[c]
    b = tb_ref[c]
    logit = jnp.where(grp == 1, b, a)
    v = g_ref[...] + logit
    m = jnp.max(v, axis=1, keepdims=True)
    cols = lax.broadcasted_iota(jnp.int32, v.shape, 1)
    masked = jnp.where(v == m, cols, BATCH)
    out_ref[...] = jnp.min(masked, axis=1, keepdims=True)


def _tc_argmax(gumbel_field, group_row, table_a, table_b):
    return pl.pallas_call(
        _argmax_body,
        in_specs=[
            pl.BlockSpec(memory_space=pltpu.MemorySpace.VMEM),
            pl.BlockSpec(memory_space=pltpu.MemorySpace.VMEM),
            pl.BlockSpec(memory_space=pltpu.MemorySpace.SMEM),
            pl.BlockSpec(memory_space=pltpu.MemorySpace.SMEM),
        ],
        out_shape=jax.ShapeDtypeStruct((BATCH, 1), jnp.int32),
    )(gumbel_field, group_row, table_a, table_b)


def _sc_gather_body(video_hbm, audio_hbm, text_hbm, tgt_hbm, grp_hbm, idx_hbm,
                    shuf_hbm, v_out, a_out, t_out, tg_out, gr_out,
                    idx_v, tgt_v, grp_v, shuf_v, fused_v, shift_v, tgo_v, gro_v,
                    vbuf0, vbuf1, abuf, tbuf,
                    gsem0, gsem1, wsem0, wsem1):
    wid = lax.axis_index("s") * NC + lax.axis_index("c")
    base = wid * BPW

    h_idx = pltpu.async_copy(idx_hbm, idx_v, gsem0)
    h_shuf = pltpu.async_copy(shuf_hbm.at[pl.ds(base, BPW)], shuf_v, gsem1)
    h_tgt = pltpu.async_copy(tgt_hbm, tgt_v, wsem0)
    h_grp = pltpu.async_copy(grp_hbm, grp_v, wsem1)
    h_idx.wait()
    h_shuf.wait()
    h_tgt.wait()
    h_grp.wait()

    lanes = lax.iota(jnp.int32, L)
    for k in range(0, BPW, L):
        sh = shuf_v[pl.ds(k, L)]
        f = plsc.load_gather(idx_v, [sh])
        fused_v[pl.ds(k, L)] = f
        tgo_v[pl.ds(k, L)] = plsc.load_gather(tgt_v, [f])
        gro_v[pl.ds(k, L)] = plsc.load_gather(grp_v, [f])
    for k in range(0, BPW, L):
        src = jnp.minimum(lanes + (k + VCHUNK), BPW - 1)
        shift_v[pl.ds(k, L)] = plsc.load_gather(fused_v, [src])

    nchunk = BPW // VCHUNK
    bufs = (vbuf0, vbuf1)
    gsems = (gsem0, gsem1)
    wsems = (wsem0, wsem1)

    def chunk_idx(c):
        if c % 2 == 0:
            return fused_v.at[pl.ds(c * VCHUNK, VCHUNK)]
        return shift_v.at[pl.ds((c - 1) * VCHUNK, VCHUNK)]

    def start_gather(c):
        return pltpu.async_copy(video_hbm.at[chunk_idx(c)], bufs[c % 2],
                                gsems[c % 2])

    def start_write(c):
        return pltpu.async_copy(bufs[c % 2],
                                v_out.at[pl.ds(base + c * VCHUNK, VCHUNK)],
                                wsems[c % 2])

    g_h = [None] * nchunk
    w_h = [None] * nchunk
    g_h[0] = start_gather(0)
    g_h[1] = start_gather(1)

    pltpu.sync_copy(tgo_v, tg_out.at[pl.ds(base, BPW)])
    pltpu.sync_copy(gro_v, gr_out.at[pl.ds(base, BPW)])
    pltpu.sync_copy(audio_hbm.at[fused_v], abuf)
    pltpu.sync_copy(abuf, a_out.at[pl.ds(base, BPW)])
    for c in range(0, BPW, TCHUNK):
        pltpu.sync_copy(text_hbm.at[fused_v.at[pl.ds(c, TCHUNK)]], tbuf)
        pltpu.sync_copy(tbuf, t_out.at[pl.ds(base + c, TCHUNK)])

    for c in range(nchunk):
        g_h[c].wait()
        w_h[c] = start_write(c)
        if c + 2 < nchunk:
            w_h[c].wait()
            g_h[c + 2] = start_gather(c + 2)
    w_h[nchunk - 2].wait()
    w_h[nchunk - 1].wait()


@functools.cache
def _sc_gather_kernel():
    mesh = plsc.VectorSubcoreMesh(core_axis_name="c", subcore_axis_name="s")
    cp = pltpu.CompilerParams()
    if "needs_layout_passes" in pltpu.CompilerParams.__dataclass_fields__:
        cp = dataclasses.replace(cp, needs_layout_passes=False)
    return pl.kernel(
        _sc_gather_body,
        compiler_params=cp,
        out_type=[
            jax.ShapeDtypeStruct((BATCH, 16, 768), jnp.float32),
            jax.ShapeDtypeStruct((BATCH, AD), jnp.float32),
            jax.ShapeDtypeStruct((BATCH, TD), jnp.float32),
            jax.ShapeDtypeStruct((BATCH,), jnp.int32),
            jax.ShapeDtypeStruct((BATCH,), jnp.int32),
        ],
        mesh=mesh,
        scratch_types=[
            pltpu.VMEM((BATCH,), jnp.int32),
            pltpu.VMEM((BATCH,), jnp.int32),
            pltpu.VMEM((BATCH,), jnp.int32),
            pltpu.VMEM((BPW,), jnp.int32),
            pltpu.VMEM((BPW,), jnp.int32),
            pltpu.VMEM((BPW,), jnp.int32),
            pltpu.VMEM((BPW,), jnp.int32),
            pltpu.VMEM((BPW,), jnp.int32),
            pltpu.VMEM((VCHUNK, 16, 768), jnp.float32),
            pltpu.VMEM((VCHUNK, 16, 768), jnp.float32),
            pltpu.VMEM((BPW, AD), jnp.float32),
            pltpu.VMEM((TCHUNK, TD), jnp.float32),
            pltpu.SemaphoreType.DMA,
            pltpu.SemaphoreType.DMA,
            pltpu.SemaphoreType.DMA,
            pltpu.SemaphoreType.DMA,
        ],
    )


@functools.cache
def _fixed_draws():
    with jax.ensure_compile_time_eval():
        key = jax.random.key(42)
        k_mult, k_perm = jax.random.split(key)
        g = np.asarray(jax.random.gumbel(k_mult, (BATCH, BATCH), jnp.float32))
        shuf = np.asarray(jax.random.permutation(k_perm, BATCH), dtype=np.int32)
        c1 = jnp.arange(BATCH + 1, dtype=jnp.int32)
        wa0 = ((BATCH - c1).astype(jnp.float32) / BATCH) ** TAU
        wa1 = (c1.astype(jnp.float32) / BATCH) ** TAU
        s = wa0 + wa1
        table_a = np.asarray(jnp.log(wa0 / s))
        table_b = np.asarray(jnp.log(wa1 / s))
    invshuf = np.argsort(shuf).astype(np.int32)
    return g, shuf, invshuf, table_a, table_b


def kernel(batch_video, batch_audio, batch_text, batch_target, batch_group):
    g_np, shuf_np, _, ta_np, tb_np = _fixed_draws()
    gumbel_field = jnp.asarray(g_np)
    shuffle_idx = jnp.asarray(shuf_np)

    indices = _tc_argmax(gumbel_field, batch_group[None, :],
                         jnp.asarray(ta_np), jnp.asarray(tb_np)).reshape(BATCH)

    v, a, t, tg, gr = _sc_gather_kernel()(batch_video, batch_audio, batch_text,
                                          batch_target, batch_group, indices,
                                          shuffle_idx)
    return (v, a, t, tg, gr)


# speedup vs baseline: 2.0686x; 1.0418x over previous
import dataclasses
import functools

import jax
import jax.numpy as jnp
import numpy as np
from jax import lax
from jax.experimental import pallas as pl
from jax.experimental.pallas import tpu as pltpu
from jax.experimental.pallas import tpu_sc as plsc

BATCH = 1024
NUM_GROUP = 2
TAU = 0.2
VD = 16 * 768
AD = 128
TD = 768

NC = 2
NS = 16
L = 16
NW = NC * NS
BPW = BATCH // NW
VCHUNK = 4
TCHUNK = 16


def _argmax_body(g_ref, group_ref, ta_ref, tb_ref, out_ref):
    grp = group_ref[...]
    c = jnp.sum(grp)
    a = ta_ref[c]
    b = tb_ref[c]
    logit = jnp.where(grp == 1, b, a)
    v = g_ref[...] + logit
    m = jnp.max(v, axis=1, keepdims=True)
    cols = lax.broadcasted_iota(jnp.int32, v.shape, 1)
    masked = jnp.where(v == m, cols, BATCH)
    out_ref[...] = jnp.min(masked, axis=1)


def _tc_argmax(gumbel_field, group_row, table_a, table_b):
    return pl.pallas_call(
        _argmax_body,
        in_specs=[
            pl.BlockSpec(memory_space=pltpu.MemorySpace.VMEM),
            pl.BlockSpec(memory_space=pltpu.MemorySpace.VMEM),
            pl.BlockSpec(memory_space=pltpu.MemorySpace.SMEM),
            pl.BlockSpec(memory_space=pltpu.MemorySpace.SMEM),
        ],
        out_shape=jax.ShapeDtypeStruct((BATCH,), jnp.int32),
    )(gumbel_field, group_row, table_a, table_b)


def _sc_gather_body(video_hbm, audio_hbm, text_hbm, tgt_hbm, grp_hbm, idx_hbm,
                    shuf_hbm, v_out, a_out, t_out, tg_out, gr_out,
                    idx_v, tgt_v, grp_v, shuf_v, fused_v, shift_v, tgo_v, gro_v,
                    vbuf0, vbuf1, abuf, tbuf,
                    gsem0, gsem1, wsem0, wsem1):
    wid = lax.axis_index("s") * NC + lax.axis_index("c")
    base = wid * BPW

    h_idx = pltpu.async_copy(idx_hbm, idx_v, gsem0)
    h_shuf = pltpu.async_copy(shuf_hbm.at[pl.ds(base, BPW)], shuf_v, gsem1)
    h_tgt = pltpu.async_copy(tgt_hbm, tgt_v, wsem0)
    h_grp = pltpu.async_copy(grp_hbm, grp_v, wsem1)
    h_idx.wait()
    h_shuf.wait()
    h_tgt.wait()
    h_grp.wait()

    lanes = lax.iota(jnp.int32, L)
    for k in range(0, BPW, L):
        sh = shuf_v[pl.ds(k, L)]
        f = plsc.load_gather(idx_v, [sh])
        fused_v[pl.ds(k, L)] = f
        tgo_v[pl.ds(k, L)] = plsc.load_gather(tgt_v, [f])
        gro_v[pl.ds(k, L)] = plsc.load_gather(grp_v, [f])
    for k in range(0, BPW, L):
        src = jnp.minimum(lanes + (k + VCHUNK), BPW - 1)
        shift_v[pl.ds(k, L)] = plsc.load_gather(fused_v, [src])

    nchunk = BPW // VCHUNK
    bufs = (vbuf0, vbuf1)
    gsems = (gsem0, gsem1)
    wsems = (wsem0, wsem1)

    def chunk_idx(c):
        if c % 2 == 0:
            return fused_v.at[pl.ds(c * VCHUNK, VCHUNK)]
        return shift_v.at[pl.ds((c - 1) * VCHUNK, VCHUNK)]

    def start_gather(c):
        return pltpu.async_copy(video_hbm.at[chunk_idx(c)], bufs[c % 2],
                                gsems[c % 2])

    def start_write(c):
        return pltpu.async_copy(bufs[c % 2],
                                v_out.at[pl.ds(base + c * VCHUNK, VCHUNK)],
                                wsems[c % 2])

    g_h = [None] * nchunk
    w_h = [None] * nchunk
    g_h[0] = start_gather(0)
    g_h[1] = start_gather(1)

    pltpu.sync_copy(tgo_v, tg_out.at[pl.ds(base, BPW)])
    pltpu.sync_copy(gro_v, gr_out.at[pl.ds(base, BPW)])
    pltpu.sync_copy(audio_hbm.at[fused_v], abuf)
    pltpu.sync_copy(abuf, a_out.at[pl.ds(base, BPW)])
    for c in range(0, BPW, TCHUNK):
        pltpu.sync_copy(text_hbm.at[fused_v.at[pl.ds(c, TCHUNK)]], tbuf)
        pltpu.sync_copy(tbuf, t_out.at[pl.ds(base + c, TCHUNK)])

    for c in range(nchunk):
        g_h[c].wait()
        w_h[c] = start_write(c)
        if c + 2 < nchunk:
            w_h[c].wait()
            g_h[c + 2] = start_gather(c + 2)
    w_h[nchunk - 2].wait()
    w_h[nchunk - 1].wait()


@functools.cache
def _sc_gather_kernel():
    mesh = plsc.VectorSubcoreMesh(core_axis_name="c", subcore_axis_name="s")
    cp = pltpu.CompilerParams()
    if "needs_layout_passes" in pltpu.CompilerParams.__dataclass_fields__:
        cp = dataclasses.replace(cp, needs_layout_passes=False)
    return pl.kernel(
        _sc_gather_body,
        compiler_params=cp,
        out_type=[
            jax.ShapeDtypeStruct((BATCH, 16, 768), jnp.float32),
            jax.ShapeDtypeStruct((BATCH, AD), jnp.float32),
            jax.ShapeDtypeStruct((BATCH, TD), jnp.float32),
            jax.ShapeDtypeStruct((BATCH,), jnp.int32),
            jax.ShapeDtypeStruct((BATCH,), jnp.int32),
        ],
        mesh=mesh,
        scratch_types=[
            pltpu.VMEM((BATCH,), jnp.int32),
            pltpu.VMEM((BATCH,), jnp.int32),
            pltpu.VMEM((BATCH,), jnp.int32),
            pltpu.VMEM((BPW,), jnp.int32),
            pltpu.VMEM((BPW,), jnp.int32),
            pltpu.VMEM((BPW,), jnp.int32),
            pltpu.VMEM((BPW,), jnp.int32),
            pltpu.VMEM((BPW,), jnp.int32),
            pltpu.VMEM((VCHUNK, 16, 768), jnp.float32),
            pltpu.VMEM((VCHUNK, 16, 768), jnp.float32),
            pltpu.VMEM((BPW, AD), jnp.float32),
            pltpu.VMEM((TCHUNK, TD), jnp.float32),
            pltpu.SemaphoreType.DMA,
            pltpu.SemaphoreType.DMA,
            pltpu.SemaphoreType.DMA,
            pltpu.SemaphoreType.DMA,
        ],
    )


@functools.cache
def _fixed_draws():
    with jax.ensure_compile_time_eval():
        key = jax.random.key(42)
        k_mult, k_perm = jax.random.split(key)
        g = np.asarray(jax.random.gumbel(k_mult, (BATCH, BATCH), jnp.float32))
        shuf = np.asarray(jax.random.permutation(k_perm, BATCH), dtype=np.int32)
        c1 = jnp.arange(BATCH + 1, dtype=jnp.int32)
        wa0 = ((BATCH - c1).astype(jnp.float32) / BATCH) ** TAU
        wa1 = (c1.astype(jnp.float32) / BATCH) ** TAU
        s = wa0 + wa1
        table_a = np.asarray(jnp.log(wa0 / s))
        table_b = np.asarray(jnp.log(wa1 / s))
    invshuf = np.argsort(shuf).astype(np.int32)
    return g, shuf, invshuf, table_a, table_b


def kernel(batch_video, batch_audio, batch_text, batch_target, batch_group):
    g_np, shuf_np, _, ta_np, tb_np = _fixed_draws()
    gumbel_field = jnp.asarray(g_np)
    shuffle_idx = jnp.asarray(shuf_np)

    indices = _tc_argmax(gumbel_field, batch_group[None, :],
                         jnp.asarray(ta_np), jnp.asarray(tb_np))

    v, a, t, tg, gr = _sc_gather_kernel()(batch_video, batch_audio, batch_text,
                                          batch_target, batch_group, indices,
                                          shuffle_idx)
    return (v, a, t, tg, gr)
